# baseline (speedup 1.0000x reference)
import functools

import jax
import jax.numpy as jnp
from jax import lax
from jax.experimental import pallas as pl
from jax.experimental.pallas import tpu as pltpu
from jax.experimental.pallas import tpu_sc as plsc

_B = 16
_M = 1000
_GENE = 978
_DIM = 16

_NC = 2
_NS = 16
_NW = _NC * _NS

_FLAT = _B * _M
_FLAT_PAD = 16384
_BW = _FLAT_PAD // _NW

_DPAD = 128


def _sc_gather(emb_pad, idx_pad):
    mesh = plsc.VectorSubcoreMesh(core_axis_name="c", subcore_axis_name="s")

    @functools.partial(
        pl.kernel,
        mesh=mesh,
        out_type=jax.ShapeDtypeStruct((_FLAT_PAD, _DPAD), jnp.float32),
        scratch_types=[
            pltpu.VMEM((_BW,), jnp.int32),
            pltpu.VMEM((_BW, _DPAD), jnp.float32),
            pltpu.SemaphoreType.DMA,
        ],
    )
    def gather_kernel(emb_hbm, idx_hbm, out_hbm, idx_v, rows_v, sem):
        wid = lax.axis_index("s") * _NC + lax.axis_index("c")
        base = wid * _BW
        pltpu.sync_copy(idx_hbm.at[pl.ds(base, _BW)], idx_v)
        pltpu.async_copy(emb_hbm.at[idx_v], rows_v, sem).wait()
        pltpu.sync_copy(rows_v, out_hbm.at[pl.ds(base, _BW)])

    return gather_kernel(emb_pad, idx_pad)


_BLK_M = 200


def _tc_body(x_ref, w_ref, b_ref, rows_ref, o_ref):
    x = x_ref[0]
    w = w_ref[...]
    cell = lax.dot_general(
        x, w, (((1,), (1,)), ((), ())), preferred_element_type=jnp.float32
    )
    cell = cell + b_ref[0][None, :]
    rows = rows_ref[:, :_DIM]
    nr = jnp.sqrt(jnp.sum(rows * rows, axis=1, keepdims=True))
    scale = jnp.minimum(1.0, 1.0 / (nr + 1e-7))
    dot = jnp.sum(cell * rows, axis=1, keepdims=True) * scale
    n1 = jnp.maximum(jnp.sqrt(jnp.sum(cell * cell, axis=1, keepdims=True)), 1e-6)
    n2 = jnp.maximum(nr * scale, 1e-6)
    o_ref[0] = dot / (n1 * n2)


def kernel(input, W, b, emb):
    idx = input[:, :, -1].astype(jnp.int32).reshape(-1)
    idx_pad = jnp.pad(idx, (0, _FLAT_PAD - _FLAT))
    emb_pad = jnp.pad(emb, ((0, 0), (0, _DPAD - _DIM)))
    rows = _sc_gather(emb_pad, idx_pad)

    w_pad = jnp.pad(W, ((0, 0), (0, 1)))
    b2 = b.reshape(1, _DIM)

    out = pl.pallas_call(
        _tc_body,
        grid=(_B, _M // _BLK_M),
        in_specs=[
            pl.BlockSpec((1, _BLK_M, _GENE + 1), lambda i, j: (i, j, 0)),
            pl.BlockSpec((_DIM, _GENE + 1), lambda i, j: (0, 0)),
            pl.BlockSpec((1, _DIM), lambda i, j: (0, 0)),
            pl.BlockSpec((_BLK_M, _DPAD), lambda i, j: (i * (_M // _BLK_M) + j, 0)),
        ],
        out_specs=pl.BlockSpec((1, _BLK_M, 1), lambda i, j: (i, j, 0)),
        out_shape=jax.ShapeDtypeStruct((_B, _M, 1), jnp.float32),
    )(input, w_pad, b2, rows)
    return out


# speedup vs baseline: 2.7099x; 2.7099x over previous
import functools

import jax
import jax.numpy as jnp
from jax import lax
from jax.experimental import pallas as pl
from jax.experimental.pallas import tpu as pltpu
from jax.experimental.pallas import tpu_sc as plsc

_B = 16
_M = 1000
_MP = 1024
_GENE = 978
_DIM = 16

_NC = 2
_NS = 16
_NW = _NC * _NS
_FLAT_PAD = _B * _MP
_BW = _FLAT_PAD // _NW

_GBLK = 89
_GSTEPS = 11


def _sc_gather_t(emb_flat, idx_pad):
    mesh = plsc.VectorSubcoreMesh(core_axis_name="c", subcore_axis_name="s")

    @functools.partial(
        pl.kernel,
        mesh=mesh,
        out_type=jax.ShapeDtypeStruct((_DIM, _FLAT_PAD), jnp.float32),
        scratch_types=[
            pltpu.VMEM((_M * _DIM,), jnp.float32),
            pltpu.VMEM((_BW,), jnp.int32),
            pltpu.VMEM((_DIM, _BW), jnp.float32),
        ],
        compiler_params=pltpu.CompilerParams(needs_layout_passes=False),
    )
    def gather_kernel(emb_hbm, idx_hbm, out_hbm, emb_v, idx_v, colbuf):
        wid = lax.axis_index("s") * _NC + lax.axis_index("c")
        base = wid * _BW
        pltpu.sync_copy(emb_hbm, emb_v)
        pltpu.sync_copy(idx_hbm.at[pl.ds(base, _BW)], idx_v)

        def body(t, carry):
            idx16 = idx_v[pl.ds(t * 16, 16)] * _DIM
            for k in range(_DIM):
                vals = plsc.load_gather(emb_v, [idx16 + k])
                colbuf[k, pl.ds(t * 16, 16)] = vals
            return carry

        lax.fori_loop(0, _BW // 16, body, 0)
        pltpu.sync_copy(colbuf, out_hbm.at[:, pl.ds(base, _BW)])

    return gather_kernel(emb_flat, idx_pad)


def _tc_cell_body(x_ref, w_ref, acc_ref):
    j = pl.program_id(0)

    @pl.when(j == 0)
    def _init():
        acc_ref[...] = jnp.zeros_like(acc_ref)

    w = w_ref[0]
    for b in range(_B):
        xb = x_ref[:, b, :]
        pm = lax.dot_general(
            w, xb, (((1,), (0,)), ((), ())), preferred_element_type=jnp.float32
        )
        acc_ref[:, b * _MP : b * _MP + _M] += pm


def _tc_combine_body(acc_ref, rows_ref, b_ref, o_ref):
    cell = acc_ref[...] + b_ref[...]
    rows = rows_ref[...]
    rssq = jnp.sum(rows * rows, axis=0, keepdims=True)
    nr = jnp.sqrt(rssq)
    scale = jnp.minimum(1.0, 1.0 / (nr + 1e-7))
    dot = jnp.sum(cell * rows, axis=0, keepdims=True) * scale
    n1 = jnp.maximum(jnp.sqrt(jnp.sum(cell * cell, axis=0, keepdims=True)), 1e-6)
    n2 = jnp.maximum(nr * scale, 1e-6)
    o_ref[0] = dot / (n1 * n2)


def kernel(input, W, b, emb):
    idx2d = input[:, :, -1].astype(jnp.int32)
    idx_pad = jnp.pad(idx2d, ((0, 0), (0, _MP - _M))).reshape(-1)
    emb_flat = emb.reshape(-1)
    rows_t = _sc_gather_t(emb_flat, idx_pad)

    x_t = jnp.transpose(input, (2, 0, 1))
    w3 = jnp.pad(W, ((0, 0), (0, 1))).reshape(_DIM, _GSTEPS, _GBLK)
    w3 = jnp.transpose(w3, (1, 0, 2))

    acc = pl.pallas_call(
        _tc_cell_body,
        grid=(_GSTEPS,),
        in_specs=[
            pl.BlockSpec((_GBLK, _B, _M), lambda j: (j, 0, 0)),
            pl.BlockSpec((1, _DIM, _GBLK), lambda j: (j, 0, 0)),
        ],
        out_specs=pl.BlockSpec((_DIM, _FLAT_PAD), lambda j: (0, 0)),
        out_shape=jax.ShapeDtypeStruct((_DIM, _FLAT_PAD), jnp.float32),
    )(x_t, w3)

    b2 = b.reshape(_DIM, 1)
    out3 = pl.pallas_call(
        _tc_combine_body,
        grid=(_B,),
        in_specs=[
            pl.BlockSpec((_DIM, _MP), lambda i: (0, i)),
            pl.BlockSpec((_DIM, _MP), lambda i: (0, i)),
            pl.BlockSpec((_DIM, 1), lambda i: (0, 0)),
        ],
        out_specs=pl.BlockSpec((1, 1, _MP), lambda i: (i, 0, 0)),
        out_shape=jax.ShapeDtypeStruct((_B, 1, _MP), jnp.float32),
    )(acc, rows_t, b2)

    return jnp.transpose(out3[:, :, :_M], (0, 2, 1))


# speedup vs baseline: 2.9809x; 1.1000x over previous
import functools

import jax
import jax.numpy as jnp
from jax import lax
from jax.experimental import pallas as pl
from jax.experimental.pallas import tpu as pltpu
from jax.experimental.pallas import tpu_sc as plsc

_B = 16
_M = 1000
_MP = 1024
_GENE = 978
_DIM = 16

_NC = 2
_NS = 16
_NW = _NC * _NS
_FLAT_PAD = _B * _MP
_BW = _FLAT_PAD // _NW

_GBLK = 89
_GSTEPS = 11


def _sc_gather_t(fidx_pad, emb_tp):
    mesh = plsc.VectorSubcoreMesh(core_axis_name="c", subcore_axis_name="s")

    @functools.partial(
        pl.kernel,
        mesh=mesh,
        out_type=jax.ShapeDtypeStruct((_DIM, _FLAT_PAD), jnp.float32),
        scratch_types=[
            pltpu.VMEM((_DIM, _MP), jnp.float32),
            pltpu.VMEM((_BW,), jnp.float32),
            pltpu.VMEM((_DIM, _BW), jnp.float32),
        ],
        compiler_params=pltpu.CompilerParams(needs_layout_passes=False),
    )
    def gather_kernel(fidx_hbm, emb_hbm, out_hbm, emb_v, fidx_v, colbuf):
        wid = lax.axis_index("s") * _NC + lax.axis_index("c")
        b = wid // 2
        mbase = (wid % 2) * _BW
        pltpu.sync_copy(emb_hbm, emb_v)
        pltpu.sync_copy(fidx_hbm.at[b, pl.ds(mbase, _BW)], fidx_v)

        def body(t, carry):
            f16 = fidx_v[pl.ds(t * 16, 16)]
            iv = jnp.clip(f16.astype(jnp.int32), 0, _M - 1)
            for k in range(_DIM):
                kvec = jnp.full((16,), k, jnp.int32)
                colbuf[k, pl.ds(t * 16, 16)] = plsc.load_gather(emb_v, [kvec, iv])
            return carry

        lax.fori_loop(0, _BW // 16, body, 0)
        pltpu.sync_copy(colbuf, out_hbm.at[:, pl.ds(wid * _BW, _BW)])

    return gather_kernel(fidx_pad, emb_tp)


def _tc_cell_body(x_ref, w_ref, acc_ref):
    j = pl.program_id(0)

    @pl.when(j == 0)
    def _init():
        acc_ref[...] = jnp.zeros_like(acc_ref)

    w = w_ref[0]
    for b in range(_B):
        xb = x_ref[:, b, :]
        pm = lax.dot_general(
            w, xb, (((1,), (0,)), ((), ())), preferred_element_type=jnp.float32
        )
        acc_ref[:, b * _MP : b * _MP + _M] += pm


def _tc_combine_body(acc_ref, rows_ref, b_ref, o_ref):
    cell = acc_ref[...] + b_ref[...]
    rows = rows_ref[...]
    rssq = jnp.sum(rows * rows, axis=0, keepdims=True)
    nr = jnp.sqrt(rssq)
    scale = jnp.minimum(1.0, 1.0 / (nr + 1e-7))
    dot = jnp.sum(cell * rows, axis=0, keepdims=True) * scale
    n1 = jnp.maximum(jnp.sqrt(jnp.sum(cell * cell, axis=0, keepdims=True)), 1e-6)
    n2 = jnp.maximum(nr * scale, 1e-6)
    o_ref[...] = dot / (n1 * n2)


def kernel(input, W, b, emb):
    x_t = jnp.transpose(input, (2, 0, 1))
    fidx_pad = jnp.pad(input[:, :, -1], ((0, 0), (0, _MP - _M)))
    emb_tp = jnp.pad(jnp.transpose(emb), ((0, 0), (0, _MP - _M)))
    rows_t = _sc_gather_t(fidx_pad, emb_tp)

    w3 = jnp.pad(W, ((0, 0), (0, 1))).reshape(_DIM, _GSTEPS, _GBLK)
    w3 = jnp.transpose(w3, (1, 0, 2))

    acc = pl.pallas_call(
        _tc_cell_body,
        grid=(_GSTEPS,),
        in_specs=[
            pl.BlockSpec((_GBLK, _B, _M), lambda j: (j, 0, 0)),
            pl.BlockSpec((1, _DIM, _GBLK), lambda j: (j, 0, 0)),
        ],
        out_specs=pl.BlockSpec((_DIM, _FLAT_PAD), lambda j: (0, 0)),
        out_shape=jax.ShapeDtypeStruct((_DIM, _FLAT_PAD), jnp.float32),
    )(x_t, w3)

    b2 = b.reshape(_DIM, 1)
    cos = pl.pallas_call(
        _tc_combine_body,
        in_specs=[
            pl.BlockSpec((_DIM, _FLAT_PAD), lambda: (0, 0)),
            pl.BlockSpec((_DIM, _FLAT_PAD), lambda: (0, 0)),
            pl.BlockSpec((_DIM, 1), lambda: (0, 0)),
        ],
        out_specs=pl.BlockSpec((1, _FLAT_PAD), lambda: (0, 0)),
        out_shape=jax.ShapeDtypeStruct((1, _FLAT_PAD), jnp.float32),
    )(acc, rows_t, b2)

    return cos.reshape(_B, _MP)[:, :_M, None]
